# baseline (speedup 1.0000x reference)
import functools

import jax
import jax.numpy as jnp
from jax import lax
from jax.experimental import pallas as pl
from jax.experimental.pallas import tpu as pltpu
from jax.experimental.pallas import tpu_sc as plsc


def _make_sc_transpose(V, R):
    info = plsc.get_sparse_core_info()
    NC = info.num_cores
    NW = NC * info.num_subcores
    VC = 512
    n_full = V // VC
    tail = V - n_full * VC
    n_iters = (n_full + NW - 1) // NW
    OB = VC * R // 128
    mesh = plsc.VectorSubcoreMesh(core_axis_name="c", subcore_axis_name="s")

    TR = tail * R // 128

    @functools.partial(
        pl.kernel,
        mesh=mesh,
        compiler_params=pltpu.CompilerParams(needs_layout_passes=False),
        out_type=jax.ShapeDtypeStruct((V * R // 128, 128), jnp.float32),
        scratch_types=[
            pltpu.VMEM((R, VC), jnp.float32),
            pltpu.VMEM((OB, 128), jnp.float32),
            pltpu.VMEM((TR, 128), jnp.float32),
        ],
    )
    def k(tableT_hbm, tailp_hbm, out_hbm, buf_in, buf_out, buf_tail):
        wid = lax.axis_index("s") * NC + lax.axis_index("c")

        def transpose_chunk(v0, vc):
            lanes = lax.iota(jnp.int32, 16)
            col_base = (lanes & 3) << 5
            row_off = lanes >> 2
            for g in range(vc // 16):
                rows_g = row_off + (4 * g)
                for f in range(R):
                    x = buf_in[f, pl.ds(16 * g, 16)]
                    plsc.store_scatter(buf_out, [rows_g, col_base + f], x)
            off = pl.multiple_of(v0 * R // 128, 16)
            pltpu.sync_copy(
                buf_out.at[pl.ds(0, vc * R // 128)],
                out_hbm.at[pl.ds(off, vc * R // 128)],
            )

        def body(it, _):
            c = it * NW + wid

            @pl.when(c < n_full)
            def _():
                v0 = pl.multiple_of(c * VC, VC)
                pltpu.sync_copy(tableT_hbm.at[:, pl.ds(v0, VC)], buf_in)
                transpose_chunk(v0, VC)
            return ()

        lax.fori_loop(0, n_iters, body, ())

        if tail:
            @pl.when(wid == NW - 1)
            def _():
                pltpu.sync_copy(tailp_hbm, buf_tail)
                pltpu.sync_copy(
                    buf_tail, out_hbm.at[pl.ds(n_full * VC * R // 128, TR)]
                )

    return k


def _make_sc_gather(V, R, N):
    info = plsc.get_sparse_core_info()
    NC, NS = info.num_cores, info.num_subcores
    NW = NC * NS
    assert N % NW == 0
    per_w = N // NW
    chunk = 3200
    assert per_w % chunk == 0
    n_chunks = per_w // chunk
    mesh = plsc.VectorSubcoreMesh(core_axis_name="c", subcore_axis_name="s")

    @functools.partial(
        pl.kernel,
        mesh=mesh,
        compiler_params=pltpu.CompilerParams(use_tc_tiling_on_sc=False),
        out_type=jax.ShapeDtypeStruct((N, R), jnp.float32),
        scratch_types=[
            pltpu.VMEM((chunk,), jnp.int32),
            pltpu.VMEM((chunk, R), jnp.float32),
            pltpu.SemaphoreType.DMA,
        ],
    )
    def k(table_hbm, idx_hbm, out_hbm, idx_v, rows_v, sem):
        wid = lax.axis_index("s") * NC + lax.axis_index("c")
        base = wid * per_w

        def body(c, _):
            off = base + c * chunk
            pltpu.sync_copy(idx_hbm.at[pl.ds(off, chunk)], idx_v)
            pltpu.async_copy(table_hbm.at[idx_v], rows_v, sem).wait()
            pltpu.sync_copy(rows_v, out_hbm.at[pl.ds(off, chunk)])
            return ()

        lax.fori_loop(0, n_chunks, body, ())

    return k


def _mm_body(x_ref, w_ref, g_ref, o_ref):
    z = jax.nn.sigmoid(g_ref[...])
    w = w_ref[...] * z.reshape(-1, 1)
    x = x_ref[...]
    R = w.shape[0]
    xs = jnp.concatenate([x[:, g * R:(g + 1) * R] for g in range(4)], axis=0)
    y = jnp.dot(xs, w, preferred_element_type=jnp.float32)
    o_ref[...] = y.reshape(4, x.shape[0], w.shape[1])


def _gated_matmul(mid_packed, right, gate_logits, block_rows=512):
    NP, RP = mid_packed.shape
    R, H = right.shape
    grid = NP // block_rows
    out = pl.pallas_call(
        _mm_body,
        grid=(grid,),
        in_specs=[
            pl.BlockSpec((block_rows, RP), lambda i: (i, 0)),
            pl.BlockSpec((R, H), lambda i: (0, 0)),
            pl.BlockSpec((1, R), lambda i: (0, 0)),
        ],
        out_specs=pl.BlockSpec((4, block_rows, H), lambda i: (0, i, 0)),
        out_shape=jax.ShapeDtypeStruct((4, NP, H), jnp.float32),
    )(mid_packed, right, gate_logits.reshape(1, R))
    return out.reshape(4 * NP, H)


def kernel(input_ids, left, right, gate_logits):
    B, L = input_ids.shape
    V, R = left.shape
    H = right.shape[1]
    N = B * L
    ids_t = input_ids.T.reshape(N)
    ids_perm = ids_t.reshape(4, N // 4).T.reshape(N)
    n_full = V // 512
    tail_packed = left[n_full * 512:, :].reshape(-1, 128)
    packT = _make_sc_transpose(V, R)(left.T, tail_packed)
    left_rm = packT.reshape(V, R)
    mid = _make_sc_gather(V, R, N)(left_rm, ids_perm)
    mid_packed = mid.reshape(N // 4, 4 * R)
    y = _gated_matmul(mid_packed, right, gate_logits)
    return y.reshape(L, B, H).transpose(1, 0, 2)


# speedup vs baseline: 1.6942x; 1.6942x over previous
import functools

import jax
import jax.numpy as jnp
from jax import lax
from jax.experimental import pallas as pl
from jax.experimental.pallas import tpu as pltpu
from jax.experimental.pallas import tpu_sc as plsc


def _make_sc_transpose(V, R):
    info = plsc.get_sparse_core_info()
    NC = info.num_cores
    NW = NC * info.num_subcores
    VC = 512
    n_full = V // VC
    tail = V - n_full * VC
    n_iters = (n_full + NW - 1) // NW
    OB = VC * R // 128
    mesh = plsc.VectorSubcoreMesh(core_axis_name="c", subcore_axis_name="s")

    TR = tail * R // 128

    @functools.partial(
        pl.kernel,
        mesh=mesh,
        compiler_params=pltpu.CompilerParams(needs_layout_passes=False),
        out_type=jax.ShapeDtypeStruct((V * R // 128, 128), jnp.float32),
        scratch_types=[
            pltpu.VMEM((R, VC), jnp.float32),
            pltpu.VMEM((OB, 128), jnp.float32),
            pltpu.VMEM((TR, 128), jnp.float32),
        ],
    )
    def k(tableT_hbm, tailp_hbm, out_hbm, buf_in, buf_out, buf_tail):
        wid = lax.axis_index("s") * NC + lax.axis_index("c")

        def transpose_chunk(v0, vc):
            lanes = lax.iota(jnp.int32, 16)
            lane_col = (lanes & 3) << 5
            row_off = lanes >> 2

            def vg_body(vg, _):
                idx_v = lanes + (vg << 4)
                rows = row_off + (vg << 2)
                for d in range(16):
                    t = (lanes + d) & 15
                    for fh in range(R // 16):
                        x = plsc.load_gather(buf_in, [t + 16 * fh, idx_v])
                        plsc.store_scatter(
                            buf_out, [rows, lane_col + t + 16 * fh], x
                        )
                return ()

            lax.fori_loop(0, vc // 16, vg_body, ())
            off = pl.multiple_of(v0 * R // 128, 16)
            pltpu.sync_copy(
                buf_out.at[pl.ds(0, vc * R // 128)],
                out_hbm.at[pl.ds(off, vc * R // 128)],
            )

        def body(it, _):
            c = it * NW + wid

            @pl.when(c < n_full)
            def _():
                v0 = pl.multiple_of(c * VC, VC)
                pltpu.sync_copy(tableT_hbm.at[:, pl.ds(v0, VC)], buf_in)
                transpose_chunk(v0, VC)
            return ()

        lax.fori_loop(0, n_iters, body, ())

        if tail:
            @pl.when(wid == NW - 1)
            def _():
                pltpu.sync_copy(tailp_hbm, buf_tail)
                pltpu.sync_copy(
                    buf_tail, out_hbm.at[pl.ds(n_full * VC * R // 128, TR)]
                )

    return k


def _make_sc_gather(V, R, N):
    info = plsc.get_sparse_core_info()
    NC, NS = info.num_cores, info.num_subcores
    NW = NC * NS
    assert N % NW == 0
    per_w = N // NW
    chunk = 3200
    assert per_w % chunk == 0
    n_chunks = per_w // chunk
    mesh = plsc.VectorSubcoreMesh(core_axis_name="c", subcore_axis_name="s")

    @functools.partial(
        pl.kernel,
        mesh=mesh,
        compiler_params=pltpu.CompilerParams(use_tc_tiling_on_sc=False),
        out_type=jax.ShapeDtypeStruct((N, R), jnp.float32),
        scratch_types=[
            pltpu.VMEM((chunk,), jnp.int32),
            pltpu.VMEM((chunk, R), jnp.float32),
            pltpu.SemaphoreType.DMA,
        ],
    )
    def k(table_hbm, idx_hbm, out_hbm, idx_v, rows_v, sem):
        wid = lax.axis_index("s") * NC + lax.axis_index("c")
        base = wid * per_w

        def body(c, _):
            off = base + c * chunk
            pltpu.sync_copy(idx_hbm.at[pl.ds(off, chunk)], idx_v)
            pltpu.async_copy(table_hbm.at[idx_v], rows_v, sem).wait()
            pltpu.sync_copy(rows_v, out_hbm.at[pl.ds(off, chunk)])
            return ()

        lax.fori_loop(0, n_chunks, body, ())

    return k


def _mm_body(x_ref, w_ref, g_ref, o_ref):
    z = jax.nn.sigmoid(g_ref[...])
    w = w_ref[...] * z.reshape(-1, 1)
    x = x_ref[...]
    R = w.shape[0]
    xs = jnp.concatenate([x[:, g * R:(g + 1) * R] for g in range(4)], axis=0)
    y = jnp.dot(xs, w, preferred_element_type=jnp.float32)
    o_ref[...] = y.reshape(4, x.shape[0], w.shape[1])


def _gated_matmul(mid_packed, right, gate_logits, block_rows=512):
    NP, RP = mid_packed.shape
    R, H = right.shape
    grid = NP // block_rows
    out = pl.pallas_call(
        _mm_body,
        grid=(grid,),
        in_specs=[
            pl.BlockSpec((block_rows, RP), lambda i: (i, 0)),
            pl.BlockSpec((R, H), lambda i: (0, 0)),
            pl.BlockSpec((1, R), lambda i: (0, 0)),
        ],
        out_specs=pl.BlockSpec((4, block_rows, H), lambda i: (0, i, 0)),
        out_shape=jax.ShapeDtypeStruct((4, NP, H), jnp.float32),
    )(mid_packed, right, gate_logits.reshape(1, R))
    return out.reshape(4 * NP, H)


def kernel(input_ids, left, right, gate_logits):
    B, L = input_ids.shape
    V, R = left.shape
    H = right.shape[1]
    N = B * L
    ids_t = input_ids.T.reshape(N)
    ids_perm = ids_t.reshape(4, N // 4).T.reshape(N)
    n_full = V // 512
    tail_packed = left[n_full * 512:, :].reshape(-1, 128)
    packT = _make_sc_transpose(V, R)(left.T, tail_packed)
    left_rm = packT.reshape(V, R)
    mid = _make_sc_gather(V, R, N)(left_rm, ids_perm)
    mid_packed = mid.reshape(N // 4, 4 * R)
    y = _gated_matmul(mid_packed, right, gate_logits)
    return y.reshape(L, B, H).transpose(1, 0, 2)


# speedup vs baseline: 2.2279x; 1.3150x over previous
import functools

import jax
import jax.numpy as jnp
from jax import lax
from jax.experimental import pallas as pl
from jax.experimental.pallas import tpu as pltpu
from jax.experimental.pallas import tpu_sc as plsc


def _make_sc_transpose(V, R):
    info = plsc.get_sparse_core_info()
    NC = info.num_cores
    NW = NC * info.num_subcores
    VC = 512
    n_full = V // VC
    tail = V - n_full * VC
    n_iters = (n_full + NW - 1) // NW
    OB = VC * R // 128
    mesh = plsc.VectorSubcoreMesh(core_axis_name="c", subcore_axis_name="s")

    TR = tail * R // 128

    @functools.partial(
        pl.kernel,
        mesh=mesh,
        compiler_params=pltpu.CompilerParams(needs_layout_passes=False),
        out_type=jax.ShapeDtypeStruct((V * R // 128, 128), jnp.float32),
        scratch_types=[
            pltpu.VMEM((R, VC), jnp.float32),
            pltpu.VMEM((R, VC), jnp.float32),
            pltpu.VMEM((OB, 128), jnp.float32),
            pltpu.VMEM((OB, 128), jnp.float32),
            pltpu.VMEM((TR, 128), jnp.float32),
            pltpu.SemaphoreType.DMA,
            pltpu.SemaphoreType.DMA,
            pltpu.SemaphoreType.DMA,
            pltpu.SemaphoreType.DMA,
        ],
    )
    def k(tableT_hbm, tailp_hbm, out_hbm,
          in0, in1, o0, o1, buf_tail, si0, si1, so0, so1):
        wid = lax.axis_index("s") * NC + lax.axis_index("c")
        buf_in = (in0, in1)
        buf_out = (o0, o1)
        in_sem = (si0, si1)
        out_sem = (so0, so1)

        def issue_in(it, p):
            c = it * NW + wid

            @pl.when(c < n_full)
            def _():
                v0 = pl.multiple_of(c * VC, VC)
                pltpu.async_copy(
                    tableT_hbm.at[:, pl.ds(v0, VC)], buf_in[p], in_sem[p]
                )

        def transpose_chunk(p):
            lanes = lax.iota(jnp.int32, 16)
            lane_col = (lanes & 3) << 5
            row_off = lanes >> 2

            def vg_body(vg, _):
                idx_v = lanes + (vg << 4)
                rows = row_off + (vg << 2)
                for d in range(16):
                    t = (lanes + d) & 15
                    for fh in range(R // 16):
                        x = plsc.load_gather(buf_in[p], [t + 16 * fh, idx_v])
                        plsc.store_scatter(
                            buf_out[p], [rows, lane_col + t + 16 * fh], x
                        )
                return ()

            lax.fori_loop(0, VC // 16, vg_body, ())

        def do_chunk(it, p):
            c = it * NW + wid

            @pl.when(c < n_full)
            def _():
                pltpu.make_async_copy(
                    tableT_hbm.at[:, pl.ds(0, VC)], buf_in[p], in_sem[p]
                ).wait()

                @pl.when(it >= 2)
                def _():
                    pltpu.make_async_copy(
                        buf_out[p], out_hbm.at[pl.ds(0, OB)], out_sem[p]
                    ).wait()

                transpose_chunk(p)
                off = pl.multiple_of(c * OB, 16)
                pltpu.async_copy(
                    buf_out[p], out_hbm.at[pl.ds(off, OB)], out_sem[p]
                )
            issue_in(it + 2, p)

        issue_in(0, 0)
        issue_in(1, 1)

        def pair_body(step, _):
            for p in range(2):
                do_chunk(2 * step + p, p)
            return ()

        lax.fori_loop(0, (n_iters + 1) // 2, pair_body, ())

        for p in range(2):
            pltpu.make_async_copy(
                buf_out[p], out_hbm.at[pl.ds(0, OB)], out_sem[p]
            ).wait()

        if tail:
            @pl.when(wid == NW - 1)
            def _():
                pltpu.sync_copy(tailp_hbm, buf_tail)
                pltpu.sync_copy(
                    buf_tail, out_hbm.at[pl.ds(n_full * VC * R // 128, TR)]
                )

    return k


def _make_sc_gather(V, R, N):
    info = plsc.get_sparse_core_info()
    NC, NS = info.num_cores, info.num_subcores
    NW = NC * NS
    assert N % NW == 0
    per_w = N // NW
    chunk = 3200
    assert per_w % chunk == 0
    n_chunks = per_w // chunk
    mesh = plsc.VectorSubcoreMesh(core_axis_name="c", subcore_axis_name="s")

    @functools.partial(
        pl.kernel,
        mesh=mesh,
        compiler_params=pltpu.CompilerParams(use_tc_tiling_on_sc=False),
        out_type=jax.ShapeDtypeStruct((N, R), jnp.float32),
        scratch_types=[
            pltpu.VMEM((chunk,), jnp.int32),
            pltpu.VMEM((chunk, R), jnp.float32),
            pltpu.SemaphoreType.DMA,
        ],
    )
    def k(table_hbm, idx_hbm, out_hbm, idx_v, rows_v, sem):
        wid = lax.axis_index("s") * NC + lax.axis_index("c")
        base = wid * per_w

        def body(c, _):
            off = base + c * chunk
            pltpu.sync_copy(idx_hbm.at[pl.ds(off, chunk)], idx_v)
            pltpu.async_copy(table_hbm.at[idx_v], rows_v, sem).wait()
            pltpu.sync_copy(rows_v, out_hbm.at[pl.ds(off, chunk)])
            return ()

        lax.fori_loop(0, n_chunks, body, ())

    return k


def _mm_body(x_ref, w_ref, g_ref, o_ref):
    z = jax.nn.sigmoid(g_ref[...])
    w = w_ref[...] * z.reshape(-1, 1)
    x = x_ref[...]
    R = w.shape[0]
    xs = jnp.concatenate([x[:, g * R:(g + 1) * R] for g in range(4)], axis=0)
    y = jnp.dot(xs, w, preferred_element_type=jnp.float32)
    o_ref[...] = y.reshape(4, x.shape[0], w.shape[1])


def _gated_matmul(mid_packed, right, gate_logits, block_rows=512):
    NP, RP = mid_packed.shape
    R, H = right.shape
    grid = NP // block_rows
    out = pl.pallas_call(
        _mm_body,
        grid=(grid,),
        in_specs=[
            pl.BlockSpec((block_rows, RP), lambda i: (i, 0)),
            pl.BlockSpec((R, H), lambda i: (0, 0)),
            pl.BlockSpec((1, R), lambda i: (0, 0)),
        ],
        out_specs=pl.BlockSpec((4, block_rows, H), lambda i: (0, i, 0)),
        out_shape=jax.ShapeDtypeStruct((4, NP, H), jnp.float32),
    )(mid_packed, right, gate_logits.reshape(1, R))
    return out.reshape(4 * NP, H)


def kernel(input_ids, left, right, gate_logits):
    B, L = input_ids.shape
    V, R = left.shape
    H = right.shape[1]
    N = B * L
    ids_t = input_ids.T.reshape(N)
    ids_perm = ids_t.reshape(4, N // 4).T.reshape(N)
    n_full = V // 512
    tail_packed = left[n_full * 512:, :].reshape(-1, 128)
    packT = _make_sc_transpose(V, R)(left.T, tail_packed)
    left_rm = packT.reshape(V, R)
    mid = _make_sc_gather(V, R, N)(left_rm, ids_perm)
    mid_packed = mid.reshape(N // 4, 4 * R)
    y = _gated_matmul(mid_packed, right, gate_logits)
    return y.reshape(L, B, H).transpose(1, 0, 2)


# speedup vs baseline: 2.4041x; 1.0791x over previous
import functools

import jax
import jax.numpy as jnp
from jax import lax
from jax.experimental import pallas as pl
from jax.experimental.pallas import tpu as pltpu
from jax.experimental.pallas import tpu_sc as plsc


def _make_sc_transpose(V, R):
    info = plsc.get_sparse_core_info()
    NC = info.num_cores
    NW = NC * info.num_subcores
    VC = 512
    n_full = V // VC
    tail = V - n_full * VC
    n_iters = (n_full + NW - 1) // NW
    OB = VC * R // 128
    mesh = plsc.VectorSubcoreMesh(core_axis_name="c", subcore_axis_name="s")

    TR = tail * R // 128

    @functools.partial(
        pl.kernel,
        mesh=mesh,
        compiler_params=pltpu.CompilerParams(needs_layout_passes=False),
        out_type=jax.ShapeDtypeStruct((V * R // 128, 128), jnp.float32),
        scratch_types=[
            pltpu.VMEM((R, VC), jnp.float32),
            pltpu.VMEM((R, VC), jnp.float32),
            pltpu.VMEM((OB, 128), jnp.float32),
            pltpu.VMEM((OB, 128), jnp.float32),
            pltpu.VMEM((TR, 128), jnp.float32),
            pltpu.SemaphoreType.DMA,
            pltpu.SemaphoreType.DMA,
            pltpu.SemaphoreType.DMA,
            pltpu.SemaphoreType.DMA,
        ],
    )
    def k(tableT_hbm, tailp_hbm, out_hbm,
          in0, in1, o0, o1, buf_tail, si0, si1, so0, so1):
        wid = lax.axis_index("s") * NC + lax.axis_index("c")
        buf_in = (in0, in1)
        buf_out = (o0, o1)
        in_sem = (si0, si1)
        out_sem = (so0, so1)

        def issue_in(it, p):
            c = it * NW + wid

            @pl.when(c < n_full)
            def _():
                v0 = pl.multiple_of(c * VC, VC)
                pltpu.async_copy(
                    tableT_hbm.at[:, pl.ds(v0, VC)], buf_in[p], in_sem[p]
                )

        def transpose_chunk(p):
            lanes = lax.iota(jnp.int32, 16)
            lane_col = (lanes & 3) << 5
            row_off = lanes >> 2

            def vg_body(vg, _):
                idx_v = lanes + (vg << 4)
                rows = row_off + (vg << 2)
                for d in range(16):
                    t = (lanes + d) & 15
                    for fh in range(R // 16):
                        x = plsc.load_gather(buf_in[p], [t + 16 * fh, idx_v])
                        plsc.store_scatter(
                            buf_out[p], [rows, lane_col + t + 16 * fh], x
                        )
                return ()

            lax.fori_loop(0, VC // 16, vg_body, (), unroll=4)

        def do_chunk(it, p):
            c = it * NW + wid

            @pl.when(c < n_full)
            def _():
                pltpu.make_async_copy(
                    tableT_hbm.at[:, pl.ds(0, VC)], buf_in[p], in_sem[p]
                ).wait()

                @pl.when(it >= 2)
                def _():
                    pltpu.make_async_copy(
                        buf_out[p], out_hbm.at[pl.ds(0, OB)], out_sem[p]
                    ).wait()

                transpose_chunk(p)
                off = pl.multiple_of(c * OB, 16)
                pltpu.async_copy(
                    buf_out[p], out_hbm.at[pl.ds(off, OB)], out_sem[p]
                )
            issue_in(it + 2, p)

        issue_in(0, 0)
        issue_in(1, 1)

        def pair_body(step, _):
            for p in range(2):
                do_chunk(2 * step + p, p)
            return ()

        lax.fori_loop(0, (n_iters + 1) // 2, pair_body, ())

        for p in range(2):
            pltpu.make_async_copy(
                buf_out[p], out_hbm.at[pl.ds(0, OB)], out_sem[p]
            ).wait()

        if tail:
            @pl.when(wid == NW - 1)
            def _():
                pltpu.sync_copy(tailp_hbm, buf_tail)
                pltpu.sync_copy(
                    buf_tail, out_hbm.at[pl.ds(n_full * VC * R // 128, TR)]
                )

    return k


def _make_sc_gather(V, R, N):
    info = plsc.get_sparse_core_info()
    NC, NS = info.num_cores, info.num_subcores
    NW = NC * NS
    assert N % NW == 0
    per_w = N // NW
    chunk = 3200
    assert per_w % chunk == 0
    n_chunks = per_w // chunk
    mesh = plsc.VectorSubcoreMesh(core_axis_name="c", subcore_axis_name="s")

    @functools.partial(
        pl.kernel,
        mesh=mesh,
        compiler_params=pltpu.CompilerParams(use_tc_tiling_on_sc=False),
        out_type=jax.ShapeDtypeStruct((N, R), jnp.float32),
        scratch_types=[
            pltpu.VMEM((chunk,), jnp.int32),
            pltpu.VMEM((chunk, R), jnp.float32),
            pltpu.SemaphoreType.DMA,
        ],
    )
    def k(table_hbm, idx_hbm, out_hbm, idx_v, rows_v, sem):
        wid = lax.axis_index("s") * NC + lax.axis_index("c")
        base = wid * per_w

        def body(c, _):
            off = base + c * chunk
            pltpu.sync_copy(idx_hbm.at[pl.ds(off, chunk)], idx_v)
            pltpu.async_copy(table_hbm.at[idx_v], rows_v, sem).wait()
            pltpu.sync_copy(rows_v, out_hbm.at[pl.ds(off, chunk)])
            return ()

        lax.fori_loop(0, n_chunks, body, ())

    return k


def _mm_body(x_ref, w_ref, g_ref, o_ref):
    z = jax.nn.sigmoid(g_ref[...])
    w = w_ref[...] * z.reshape(-1, 1)
    x = x_ref[...]
    R = w.shape[0]
    xs = jnp.concatenate([x[:, g * R:(g + 1) * R] for g in range(4)], axis=0)
    y = jnp.dot(xs, w, preferred_element_type=jnp.float32)
    o_ref[...] = y.reshape(4, x.shape[0], w.shape[1])


def _gated_matmul(mid_packed, right, gate_logits, block_rows=512):
    NP, RP = mid_packed.shape
    R, H = right.shape
    grid = NP // block_rows
    out = pl.pallas_call(
        _mm_body,
        grid=(grid,),
        in_specs=[
            pl.BlockSpec((block_rows, RP), lambda i: (i, 0)),
            pl.BlockSpec((R, H), lambda i: (0, 0)),
            pl.BlockSpec((1, R), lambda i: (0, 0)),
        ],
        out_specs=pl.BlockSpec((4, block_rows, H), lambda i: (0, i, 0)),
        out_shape=jax.ShapeDtypeStruct((4, NP, H), jnp.float32),
    )(mid_packed, right, gate_logits.reshape(1, R))
    return out.reshape(4 * NP, H)


def kernel(input_ids, left, right, gate_logits):
    B, L = input_ids.shape
    V, R = left.shape
    H = right.shape[1]
    N = B * L
    ids_t = input_ids.T.reshape(N)
    ids_perm = ids_t.reshape(4, N // 4).T.reshape(N)
    n_full = V // 512
    tail_packed = left[n_full * 512:, :].reshape(-1, 128)
    packT = _make_sc_transpose(V, R)(left.T, tail_packed)
    left_rm = packT.reshape(V, R)
    mid = _make_sc_gather(V, R, N)(left_rm, ids_perm)
    mid_packed = mid.reshape(N // 4, 4 * R)
    y = _gated_matmul(mid_packed, right, gate_logits)
    return y.reshape(L, B, H).transpose(1, 0, 2)


# speedup vs baseline: 2.5061x; 1.0424x over previous
import functools

import jax
import jax.numpy as jnp
from jax import lax
from jax.experimental import pallas as pl
from jax.experimental.pallas import tpu as pltpu
from jax.experimental.pallas import tpu_sc as plsc


_TRANSPOSE_VC = 768


def _make_sc_transpose(V, R):
    info = plsc.get_sparse_core_info()
    NC = info.num_cores
    NW = NC * info.num_subcores
    VC = _TRANSPOSE_VC
    n_full = V // VC
    tail = V - n_full * VC
    n_iters = (n_full + NW - 1) // NW
    OB = VC * R // 128
    mesh = plsc.VectorSubcoreMesh(core_axis_name="c", subcore_axis_name="s")

    TR = tail * R // 128

    @functools.partial(
        pl.kernel,
        mesh=mesh,
        compiler_params=pltpu.CompilerParams(needs_layout_passes=False),
        out_type=jax.ShapeDtypeStruct((V * R // 128, 128), jnp.float32),
        scratch_types=[
            pltpu.VMEM((R, VC), jnp.float32),
            pltpu.VMEM((R, VC), jnp.float32),
            pltpu.VMEM((OB, 128), jnp.float32),
            pltpu.VMEM((OB, 128), jnp.float32),
            pltpu.VMEM((TR, 128), jnp.float32),
            pltpu.SemaphoreType.DMA,
            pltpu.SemaphoreType.DMA,
            pltpu.SemaphoreType.DMA,
            pltpu.SemaphoreType.DMA,
        ],
    )
    def k(tableT_hbm, tailp_hbm, out_hbm,
          in0, in1, o0, o1, buf_tail, si0, si1, so0, so1):
        wid = lax.axis_index("s") * NC + lax.axis_index("c")
        buf_in = (in0, in1)
        buf_out = (o0, o1)
        in_sem = (si0, si1)
        out_sem = (so0, so1)

        def issue_in(it, p):
            c = it * NW + wid

            @pl.when(c < n_full)
            def _():
                v0 = pl.multiple_of(c * VC, VC)
                pltpu.async_copy(
                    tableT_hbm.at[:, pl.ds(v0, VC)], buf_in[p], in_sem[p]
                )

        def transpose_chunk(p):
            lanes = lax.iota(jnp.int32, 16)
            lane_col = (lanes & 3) << 5
            row_off = lanes >> 2

            def vg_body(vg, _):
                idx_v = lanes + (vg << 4)
                rows = row_off + (vg << 2)
                for d in range(16):
                    t = (lanes + d) & 15
                    for fh in range(R // 16):
                        x = plsc.load_gather(buf_in[p], [t + 16 * fh, idx_v])
                        plsc.store_scatter(
                            buf_out[p], [rows, lane_col + t + 16 * fh], x
                        )
                return ()

            lax.fori_loop(0, VC // 16, vg_body, (), unroll=4)

        def do_chunk(it, p):
            c = it * NW + wid

            @pl.when(c < n_full)
            def _():
                pltpu.make_async_copy(
                    tableT_hbm.at[:, pl.ds(0, VC)], buf_in[p], in_sem[p]
                ).wait()

                @pl.when(it >= 2)
                def _():
                    pltpu.make_async_copy(
                        buf_out[p], out_hbm.at[pl.ds(0, OB)], out_sem[p]
                    ).wait()

                transpose_chunk(p)
                off = pl.multiple_of(c * OB, 16)
                pltpu.async_copy(
                    buf_out[p], out_hbm.at[pl.ds(off, OB)], out_sem[p]
                )
            issue_in(it + 2, p)

        issue_in(0, 0)
        issue_in(1, 1)

        def pair_body(step, _):
            for p in range(2):
                do_chunk(2 * step + p, p)
            return ()

        lax.fori_loop(0, (n_iters + 1) // 2, pair_body, ())

        for p in range(2):
            pltpu.make_async_copy(
                buf_out[p], out_hbm.at[pl.ds(0, OB)], out_sem[p]
            ).wait()

        if tail:
            @pl.when(wid == NW - 1)
            def _():
                pltpu.sync_copy(tailp_hbm, buf_tail)
                pltpu.sync_copy(
                    buf_tail, out_hbm.at[pl.ds(n_full * VC * R // 128, TR)]
                )

    return k


def _make_sc_gather(V, R, N):
    info = plsc.get_sparse_core_info()
    NC, NS = info.num_cores, info.num_subcores
    NW = NC * NS
    assert N % NW == 0
    per_w = N // NW
    chunk = 3200
    assert per_w % chunk == 0
    n_chunks = per_w // chunk
    mesh = plsc.VectorSubcoreMesh(core_axis_name="c", subcore_axis_name="s")

    @functools.partial(
        pl.kernel,
        mesh=mesh,
        compiler_params=pltpu.CompilerParams(use_tc_tiling_on_sc=False),
        out_type=jax.ShapeDtypeStruct((N, R), jnp.float32),
        scratch_types=[
            pltpu.VMEM((chunk,), jnp.int32),
            pltpu.VMEM((chunk, R), jnp.float32),
            pltpu.SemaphoreType.DMA,
        ],
    )
    def k(table_hbm, idx_hbm, out_hbm, idx_v, rows_v, sem):
        wid = lax.axis_index("s") * NC + lax.axis_index("c")
        base = wid * per_w

        def body(c, _):
            off = base + c * chunk
            pltpu.sync_copy(idx_hbm.at[pl.ds(off, chunk)], idx_v)
            pltpu.async_copy(table_hbm.at[idx_v], rows_v, sem).wait()
            pltpu.sync_copy(rows_v, out_hbm.at[pl.ds(off, chunk)])
            return ()

        lax.fori_loop(0, n_chunks, body, ())

    return k


def _mm_body(x_ref, w_ref, g_ref, o_ref):
    z = jax.nn.sigmoid(g_ref[...])
    w = w_ref[...] * z.reshape(-1, 1)
    x = x_ref[...]
    R = w.shape[0]
    xs = jnp.concatenate([x[:, g * R:(g + 1) * R] for g in range(4)], axis=0)
    y = jnp.dot(xs, w, preferred_element_type=jnp.float32)
    o_ref[...] = y.reshape(4, x.shape[0], w.shape[1])


def _gated_matmul(mid_packed, right, gate_logits, block_rows=1024):
    NP, RP = mid_packed.shape
    R, H = right.shape
    grid = NP // block_rows
    out = pl.pallas_call(
        _mm_body,
        grid=(grid,),
        in_specs=[
            pl.BlockSpec((block_rows, RP), lambda i: (i, 0)),
            pl.BlockSpec((R, H), lambda i: (0, 0)),
            pl.BlockSpec((1, R), lambda i: (0, 0)),
        ],
        out_specs=pl.BlockSpec((4, block_rows, H), lambda i: (0, i, 0)),
        out_shape=jax.ShapeDtypeStruct((4, NP, H), jnp.float32),
    )(mid_packed, right, gate_logits.reshape(1, R))
    return out.reshape(4 * NP, H)


def kernel(input_ids, left, right, gate_logits):
    B, L = input_ids.shape
    V, R = left.shape
    H = right.shape[1]
    N = B * L
    ids_t = input_ids.T.reshape(N)
    ids_perm = ids_t.reshape(4, N // 4).T.reshape(N)
    n_full = V // _TRANSPOSE_VC
    tail_packed = left[n_full * _TRANSPOSE_VC:, :].reshape(-1, 128)
    packT = _make_sc_transpose(V, R)(left.T, tail_packed)
    left_rm = packT.reshape(V, R)
    mid = _make_sc_gather(V, R, N)(left_rm, ids_perm)
    mid_packed = mid.reshape(N // 4, 4 * R)
    y = _gated_matmul(mid_packed, right, gate_logits)
    return y.reshape(L, B, H).transpose(1, 0, 2)


# speedup vs baseline: 2.6013x; 1.0380x over previous
import functools

import jax
import jax.numpy as jnp
from jax import lax
from jax.experimental import pallas as pl
from jax.experimental.pallas import tpu as pltpu
from jax.experimental.pallas import tpu_sc as plsc


_TRANSPOSE_VC = 768


def _make_sc_transpose(V, R):
    info = plsc.get_sparse_core_info()
    NC = info.num_cores
    NW = NC * info.num_subcores
    VC = _TRANSPOSE_VC
    n_full = V // VC
    tail = V - n_full * VC
    n_iters = (n_full + NW - 1) // NW
    OB = VC * R // 128
    mesh = plsc.VectorSubcoreMesh(core_axis_name="c", subcore_axis_name="s")

    TR = tail * R // 128

    @functools.partial(
        pl.kernel,
        mesh=mesh,
        compiler_params=pltpu.CompilerParams(needs_layout_passes=False),
        out_type=jax.ShapeDtypeStruct((V * R // 128, 128), jnp.float32),
        scratch_types=[
            pltpu.VMEM((R, VC), jnp.float32),
            pltpu.VMEM((R, VC), jnp.float32),
            pltpu.VMEM((OB, 128), jnp.float32),
            pltpu.VMEM((OB, 128), jnp.float32),
            pltpu.VMEM((TR, 128), jnp.float32),
            pltpu.SemaphoreType.DMA,
            pltpu.SemaphoreType.DMA,
            pltpu.SemaphoreType.DMA,
            pltpu.SemaphoreType.DMA,
        ],
    )
    def k(tableT_hbm, tailp_hbm, out_hbm,
          in0, in1, o0, o1, buf_tail, si0, si1, so0, so1):
        wid = lax.axis_index("s") * NC + lax.axis_index("c")
        buf_in = (in0, in1)
        buf_out = (o0, o1)
        in_sem = (si0, si1)
        out_sem = (so0, so1)

        def issue_in(it, p):
            c = it * NW + wid

            @pl.when(c < n_full)
            def _():
                v0 = pl.multiple_of(c * VC, VC)
                pltpu.async_copy(
                    tableT_hbm.at[:, pl.ds(v0, VC)], buf_in[p], in_sem[p]
                )

        def transpose_chunk(p):
            lanes = lax.iota(jnp.int32, 16)
            lane_col = (lanes & 3) << 5
            row_off = lanes >> 2

            def d_body(d, _):
                t = (lanes + d) & 15
                fi = [t + 16 * fh for fh in range(R // 16)]
                ci = [lane_col + t + 16 * fh for fh in range(R // 16)]
                for vg in range(VC // 16):
                    idx_v = lanes + (vg << 4)
                    rows = row_off + (vg << 2)
                    for fh in range(R // 16):
                        x = plsc.load_gather(buf_in[p], [fi[fh], idx_v])
                        plsc.store_scatter(buf_out[p], [rows, ci[fh]], x)
                return ()

            lax.fori_loop(0, 16, d_body, ())

        def do_chunk(it, p):
            c = it * NW + wid

            @pl.when(c < n_full)
            def _():
                pltpu.make_async_copy(
                    tableT_hbm.at[:, pl.ds(0, VC)], buf_in[p], in_sem[p]
                ).wait()

                @pl.when(it >= 2)
                def _():
                    pltpu.make_async_copy(
                        buf_out[p], out_hbm.at[pl.ds(0, OB)], out_sem[p]
                    ).wait()

                transpose_chunk(p)
                off = pl.multiple_of(c * OB, 16)
                pltpu.async_copy(
                    buf_out[p], out_hbm.at[pl.ds(off, OB)], out_sem[p]
                )
            issue_in(it + 2, p)

        issue_in(0, 0)
        issue_in(1, 1)

        def pair_body(step, _):
            for p in range(2):
                do_chunk(2 * step + p, p)
            return ()

        lax.fori_loop(0, (n_iters + 1) // 2, pair_body, ())

        for p in range(2):
            pltpu.make_async_copy(
                buf_out[p], out_hbm.at[pl.ds(0, OB)], out_sem[p]
            ).wait()

        if tail:
            @pl.when(wid == NW - 1)
            def _():
                pltpu.sync_copy(tailp_hbm, buf_tail)
                pltpu.sync_copy(
                    buf_tail, out_hbm.at[pl.ds(n_full * VC * R // 128, TR)]
                )

    return k


def _make_sc_gather(V, R, N):
    info = plsc.get_sparse_core_info()
    NC, NS = info.num_cores, info.num_subcores
    NW = NC * NS
    assert N % NW == 0
    per_w = N // NW
    chunk = 3200
    assert per_w % chunk == 0
    n_chunks = per_w // chunk
    mesh = plsc.VectorSubcoreMesh(core_axis_name="c", subcore_axis_name="s")

    @functools.partial(
        pl.kernel,
        mesh=mesh,
        compiler_params=pltpu.CompilerParams(use_tc_tiling_on_sc=False),
        out_type=jax.ShapeDtypeStruct((N, R), jnp.float32),
        scratch_types=[
            pltpu.VMEM((chunk,), jnp.int32),
            pltpu.VMEM((chunk, R), jnp.float32),
            pltpu.SemaphoreType.DMA,
        ],
    )
    def k(table_hbm, idx_hbm, out_hbm, idx_v, rows_v, sem):
        wid = lax.axis_index("s") * NC + lax.axis_index("c")
        base = wid * per_w

        def body(c, _):
            off = base + c * chunk
            pltpu.sync_copy(idx_hbm.at[pl.ds(off, chunk)], idx_v)
            pltpu.async_copy(table_hbm.at[idx_v], rows_v, sem).wait()
            pltpu.sync_copy(rows_v, out_hbm.at[pl.ds(off, chunk)])
            return ()

        lax.fori_loop(0, n_chunks, body, ())

    return k


def _mm_body(x_ref, w_ref, g_ref, o_ref):
    z = jax.nn.sigmoid(g_ref[...])
    w = w_ref[...] * z.reshape(-1, 1)
    x = x_ref[...]
    R = w.shape[0]
    xs = jnp.concatenate([x[:, g * R:(g + 1) * R] for g in range(4)], axis=0)
    y = jnp.dot(xs, w, preferred_element_type=jnp.float32)
    o_ref[...] = y.reshape(4, x.shape[0], w.shape[1])


def _gated_matmul(mid_packed, right, gate_logits, block_rows=1024):
    NP, RP = mid_packed.shape
    R, H = right.shape
    grid = NP // block_rows
    out = pl.pallas_call(
        _mm_body,
        grid=(grid,),
        in_specs=[
            pl.BlockSpec((block_rows, RP), lambda i: (i, 0)),
            pl.BlockSpec((R, H), lambda i: (0, 0)),
            pl.BlockSpec((1, R), lambda i: (0, 0)),
        ],
        out_specs=pl.BlockSpec((4, block_rows, H), lambda i: (0, i, 0)),
        out_shape=jax.ShapeDtypeStruct((4, NP, H), jnp.float32),
    )(mid_packed, right, gate_logits.reshape(1, R))
    return out.reshape(4 * NP, H)


def kernel(input_ids, left, right, gate_logits):
    B, L = input_ids.shape
    V, R = left.shape
    H = right.shape[1]
    N = B * L
    ids_t = input_ids.T.reshape(N)
    ids_perm = ids_t.reshape(4, N // 4).T.reshape(N)
    n_full = V // _TRANSPOSE_VC
    tail_packed = left[n_full * _TRANSPOSE_VC:, :].reshape(-1, 128)
    packT = _make_sc_transpose(V, R)(left.T, tail_packed)
    left_rm = packT.reshape(V, R)
    mid = _make_sc_gather(V, R, N)(left_rm, ids_perm)
    mid_packed = mid.reshape(N // 4, 4 * R)
    y = _gated_matmul(mid_packed, right, gate_logits)
    return y.reshape(L, B, H).transpose(1, 0, 2)
